# baseline (speedup 1.0000x reference)
import jax
import jax.numpy as jnp
from jax.experimental import pallas as pl
from jax.experimental.pallas import tpu as pltpu

D_MODEL = 1024
BATCH = 16384
BM = 4096


def _tc_body(ids_ref, bbox_ref, wt_ref, b_ref, view_ref, kind_ref, out_ref):
    view_id = ids_ref[0]
    kind_id = ids_ref[1]
    vt = view_ref[...]
    kt = kind_ref[...]
    vsel = (jax.lax.broadcasted_iota(jnp.int32, vt.shape, 0) == view_id)
    ksel = (jax.lax.broadcasted_iota(jnp.int32, kt.shape, 0) == kind_id)
    vrow = jnp.sum(jnp.where(vsel, vt, 0.0), axis=0)
    krow = jnp.sum(jnp.where(ksel, kt, 0.0), axis=0)
    bias = b_ref[...] + vrow + krow
    acc = jnp.dot(bbox_ref[...], wt_ref[...], preferred_element_type=jnp.float32)
    out_ref[...] = acc + bias[None, :]


def kernel(bbox, kind_id, view_id, W_bbox, b_bbox, view_table, kind_table):
    bb = bbox if bbox.ndim > 1 else bbox[None, :]
    m = bb.shape[0]
    ids = jnp.stack([jnp.asarray(view_id, jnp.int32), jnp.asarray(kind_id, jnp.int32)])
    wt = W_bbox.T
    bm = BM if m % BM == 0 else m
    grid = (m // bm,)
    out = pl.pallas_call(
        _tc_body,
        grid=grid,
        in_specs=[
            pl.BlockSpec(memory_space=pltpu.SMEM),
            pl.BlockSpec((bm, 4), lambda i: (i, 0)),
            pl.BlockSpec((4, D_MODEL), lambda i: (0, 0)),
            pl.BlockSpec((D_MODEL,), lambda i: (0,)),
            pl.BlockSpec((4, D_MODEL), lambda i: (0, 0)),
            pl.BlockSpec((2, D_MODEL), lambda i: (0, 0)),
        ],
        out_specs=pl.BlockSpec((bm, D_MODEL), lambda i: (i, 0)),
        out_shape=jax.ShapeDtypeStruct((m, D_MODEL), jnp.float32),
        compiler_params=pltpu.CompilerParams(
            dimension_semantics=("arbitrary",),
        ),
    )(ids, bb, wt, b_bbox, view_table, kind_table)
    if out.shape[0] == 1:
        out = out[0]
    return out


# speedup vs baseline: 1.0313x; 1.0313x over previous
import jax
import jax.numpy as jnp
from jax.experimental import pallas as pl
from jax.experimental.pallas import tpu as pltpu

D_MODEL = 1024
BATCH = 16384
BM = 2048


def _tc_body(ids_ref, bbox_ref, wt_ref, b_ref, view_ref, kind_ref, out_ref):
    view_id = ids_ref[0]
    kind_id = ids_ref[1]
    vt = view_ref[...]
    kt = kind_ref[...]
    vsel = (jax.lax.broadcasted_iota(jnp.int32, vt.shape, 0) == view_id)
    ksel = (jax.lax.broadcasted_iota(jnp.int32, kt.shape, 0) == kind_id)
    vrow = jnp.sum(jnp.where(vsel, vt, 0.0), axis=0)
    krow = jnp.sum(jnp.where(ksel, kt, 0.0), axis=0)
    bias = b_ref[...] + vrow + krow
    acc = jnp.dot(bbox_ref[...], wt_ref[...], preferred_element_type=jnp.float32)
    out_ref[...] = acc + bias[None, :]


def kernel(bbox, kind_id, view_id, W_bbox, b_bbox, view_table, kind_table):
    bb = bbox if bbox.ndim > 1 else bbox[None, :]
    m = bb.shape[0]
    ids = jnp.stack([jnp.asarray(view_id, jnp.int32), jnp.asarray(kind_id, jnp.int32)])
    wt = W_bbox.T
    bm = BM if m % BM == 0 else m
    grid = (m // bm,)
    out = pl.pallas_call(
        _tc_body,
        grid=grid,
        in_specs=[
            pl.BlockSpec(memory_space=pltpu.SMEM),
            pl.BlockSpec((bm, 4), lambda i: (i, 0)),
            pl.BlockSpec((4, D_MODEL), lambda i: (0, 0)),
            pl.BlockSpec((D_MODEL,), lambda i: (0,)),
            pl.BlockSpec((4, D_MODEL), lambda i: (0, 0)),
            pl.BlockSpec((2, D_MODEL), lambda i: (0, 0)),
        ],
        out_specs=pl.BlockSpec((bm, D_MODEL), lambda i: (i, 0)),
        out_shape=jax.ShapeDtypeStruct((m, D_MODEL), jnp.float32),
        compiler_params=pltpu.CompilerParams(
            dimension_semantics=("parallel",),
        ),
    )(ids, bb, wt, b_bbox, view_table, kind_table)
    if out.shape[0] == 1:
        out = out[0]
    return out
